# baseline (speedup 1.0000x reference)
import functools

import jax
import jax.numpy as jnp
from jax import lax
from jax.experimental import pallas as pl
from jax.experimental.pallas import tpu as pltpu
from jax.experimental.pallas import tpu_sc as plsc

_NUM_CORES = 2
_NUM_SUBCORES = 16
_NUM_WORKERS = _NUM_CORES * _NUM_SUBCORES
_CHUNK = 128


def _sc_gather(idx3d, table):
    n_chunks = idx3d.shape[1]
    b_per_w = n_chunks * _CHUNK
    d = table.shape[1]
    mesh = plsc.VectorSubcoreMesh(core_axis_name="c", subcore_axis_name="s")

    @functools.partial(
        pl.kernel,
        out_type=jax.ShapeDtypeStruct((_NUM_WORKERS * b_per_w, d), jnp.float32),
        mesh=mesh,
        scratch_types=[
            pltpu.VMEM((n_chunks, _CHUNK), jnp.int32),
            pltpu.VMEM((b_per_w, d), jnp.float32),
            pltpu.SemaphoreType.DMA,
        ],
        compiler_params=pltpu.CompilerParams(use_tc_tiling_on_sc=False),
    )
    def k(idx_hbm, table_hbm, out_hbm, idx_v, rows_v, sem):
        wid = lax.axis_index("s") * _NUM_CORES + lax.axis_index("c")
        pltpu.sync_copy(idx_hbm.at[wid], idx_v)

        def fire(j, carry):
            pltpu.async_copy(
                table_hbm.at[idx_v.at[j]],
                rows_v.at[pl.ds(j * _CHUNK, _CHUNK)],
                sem,
            )
            return carry

        lax.fori_loop(0, n_chunks, fire, 0)

        def drain(j, carry):
            pltpu.make_async_copy(
                table_hbm.at[idx_v.at[0]],
                rows_v.at[pl.ds(0, _CHUNK)],
                sem,
            ).wait()
            return carry

        lax.fori_loop(0, n_chunks, drain, 0)
        pltpu.sync_copy(rows_v, out_hbm.at[pl.ds(wid * b_per_w, b_per_w)])

    return k(idx3d, table)


def kernel(sparse_input, table):
    batch, n_fields = sparse_input.shape
    total = batch * n_fields
    n_chunks = total // (_NUM_WORKERS * _CHUNK)
    idx3d = sparse_input.astype(jnp.int32).reshape(_NUM_WORKERS, n_chunks, _CHUNK)
    out_flat = _sc_gather(idx3d, table)
    return out_flat.reshape(batch, n_fields, table.shape[1])


# speedup vs baseline: 1.4750x; 1.4750x over previous
import functools

import jax
import jax.numpy as jnp
from jax import lax
from jax.experimental import pallas as pl
from jax.experimental.pallas import tpu as pltpu
from jax.experimental.pallas import tpu_sc as plsc

_NC = 2
_NS = 16
_NW = _NC * _NS
_R = 8
_L = 4


def _sc_gather(idx3d, table, batch, n_fields):
    b_per_w = idx3d.shape[1]
    fpad = idx3d.shape[2]
    d = table.shape[1]
    mesh = plsc.VectorSubcoreMesh(core_axis_name="c", subcore_axis_name="s")

    @functools.partial(
        pl.kernel,
        out_type=jax.ShapeDtypeStruct((batch, n_fields, d), jnp.float32),
        mesh=mesh,
        scratch_types=[
            pltpu.VMEM((b_per_w, fpad), jnp.int32),
            pltpu.VMEM((_R, n_fields, d), jnp.float32),
            pltpu.VMEM((d,), jnp.float32),
            pltpu.SemaphoreType.DMA,
            pltpu.SemaphoreType.DMA,
        ],
    )
    def k(idx_hbm, table_hbm, out_hbm, idx_v, ring_v, dummy_v, sem_g, sem_w):
        wid = lax.axis_index("s") * _NC + lax.axis_index("c")
        pltpu.sync_copy(idx_hbm.at[wid], idx_v)
        b0 = wid * b_per_w

        def wait_write(b):
            pltpu.make_async_copy(
                ring_v.at[b & (_R - 1)], out_hbm.at[b0 + b], sem_w
            ).wait()

        def fire_write(b):
            pltpu.async_copy(
                ring_v.at[b & (_R - 1)], out_hbm.at[b0 + b], sem_w
            )

        def wait_gathers():
            def w1(j, carry):
                pltpu.make_async_copy(table_hbm.at[0], dummy_v, sem_g).wait()
                return carry

            lax.fori_loop(0, n_fields, w1, 0)

        def fire_gathers(b):
            slot = b & (_R - 1)
            vec_a = idx_v[b, pl.ds(0, 16)]
            vec_b = idx_v[b, pl.ds(16, 16)]
            for f in range(n_fields):
                r = vec_a[f] if f < 16 else vec_b[f - 16]
                pltpu.async_copy(table_hbm.at[r], ring_v.at[slot, f], sem_g)

        def body(b, carry):
            @pl.when(b >= _R)
            def _():
                wait_write(b - _R)

            fire_gathers(b)

            @pl.when(b >= _L)
            def _():
                wait_gathers()
                fire_write(b - _L)

            return carry

        lax.fori_loop(0, b_per_w, body, 0)

        def tail(t, carry):
            wait_gathers()
            fire_write(b_per_w - _L + t)
            return carry

        lax.fori_loop(0, _L, tail, 0)

        def tail_w(t, carry):
            wait_write(b_per_w - _R + t)
            return carry

        lax.fori_loop(0, _R, tail_w, 0)

    return k(idx3d, table)


def kernel(sparse_input, table):
    batch, n_fields = sparse_input.shape
    b_per_w = batch // _NW
    idx_pad = jnp.pad(sparse_input.astype(jnp.int32), ((0, 0), (0, 32 - n_fields)))
    idx3d = idx_pad.reshape(_NW, b_per_w, 32)
    return _sc_gather(idx3d, table, batch, n_fields)
